# baseline (speedup 1.0000x reference)
import functools

import jax
import jax.numpy as jnp
import numpy as np
from jax import lax
from jax.experimental import pallas as pl
from jax.experimental.pallas import tpu as pltpu
from jax.experimental.pallas import tpu_sc as plsc

_DIM = 256
_NNZ = _DIM * (_DIM + 1) // 2
_BATCH = 1024
_NTILES = 32
_PER_W = _BATCH // _NTILES
_HALF = _DIM // 2
_CNT_A = sum(_DIM - i for i in range(_HALF))
_CNT_B = _NNZ - _CNT_A


def _packed_triu_idx() -> np.ndarray:
    i, j = np.triu_indices(_DIM)
    a = i < _HALF
    idx_a = (i[a] << 8) | j[a]
    idx_b = ((i[~a] - _HALF) << 8) | (j[~a] - _HALF)
    return np.concatenate([idx_a, idx_b]).astype(np.int32)


_IDX = _packed_triu_idx()


def _tri_body(
    x_hbm, idx_hbm, out_hbm,
    idx_v, buf_a, buf_b, out_v,
    sem_a, sem_b, sem_out, sem_out2,
):
    nc = 2
    wid = lax.axis_index("s") * nc + lax.axis_index("c")
    b0 = wid * _PER_W

    def stage_a(b):
        return pltpu.make_async_copy(
            x_hbm.at[b, pl.ds(0, _HALF)], buf_a, sem_a
        )

    def stage_b(b):
        return pltpu.make_async_copy(
            x_hbm.at[b, pl.ds(_HALF, _HALF), pl.ds(_HALF, _HALF)],
            buf_b,
            sem_b,
        )

    def out_copy_a(b):
        return pltpu.make_async_copy(
            out_v.at[pl.ds(0, _CNT_A)],
            out_hbm.at[pl.ds(b * _NNZ, _CNT_A)],
            sem_out,
        )

    def out_copy_b(b):
        return pltpu.make_async_copy(
            out_v.at[pl.ds(_CNT_A, _CNT_B)],
            out_hbm.at[pl.ds(b * _NNZ + _CNT_A, _CNT_B)],
            sem_out2,
        )

    def gather_span(buf, chunk0, nchunks):
        group = 4

        @plsc.parallel_loop(0, nchunks // group, unroll=8)
        def _(cg):
            base = (chunk0 + cg * group) * 16
            ivs = [idx_v[pl.ds(base + 16 * u, 16)] for u in range(group)]
            rs = [lax.shift_right_logical(iv, 8) for iv in ivs]
            cs = [lax.bitwise_and(iv, 255) for iv in ivs]
            gs = [
                plsc.load_gather(buf, [rs[u], cs[u]]) for u in range(group)
            ]
            for u in range(group):
                out_v[pl.ds(base + 16 * u, 16)] = gs[u]

    stage_a(b0).start()
    stage_b(b0).start()
    pltpu.sync_copy(idx_hbm, idx_v)

    def batch_body(bl, carry):
        b = b0 + bl
        stage_a(b).wait()

        @pl.when(bl > 0)
        def _():
            out_copy_a(b - 1).wait()

        gather_span(buf_a, 0, _CNT_A // 16)

        @pl.when(bl < _PER_W - 1)
        def _():
            stage_a(b + 1).start()

        out_copy_a(b).start()
        stage_b(b).wait()

        @pl.when(bl > 0)
        def _():
            out_copy_b(b - 1).wait()

        gather_span(buf_b, _CNT_A // 16, _CNT_B // 16)

        @pl.when(bl < _PER_W - 1)
        def _():
            stage_b(b + 1).start()

        out_copy_b(b).start()
        return carry

    lax.fori_loop(0, _PER_W, batch_body, 0)
    out_copy_a(b0 + _PER_W - 1).wait()
    out_copy_b(b0 + _PER_W - 1).wait()


@jax.jit
def _tritovec(x, idx):
    mesh = plsc.VectorSubcoreMesh(core_axis_name="c", subcore_axis_name="s")
    fn = functools.partial(
        pl.kernel,
        mesh=mesh,
        out_type=jax.ShapeDtypeStruct((_BATCH * _NNZ,), jnp.float32),
        scratch_types=[
            pltpu.VMEM((_NNZ,), jnp.int32),
            pltpu.VMEM((_HALF, _DIM), jnp.float32),
            pltpu.VMEM((_HALF, _HALF), jnp.float32),
            pltpu.VMEM((_NNZ,), jnp.float32),
            pltpu.SemaphoreType.DMA,
            pltpu.SemaphoreType.DMA,
            pltpu.SemaphoreType.DMA,
            pltpu.SemaphoreType.DMA,
        ],
        compiler_params=pltpu.CompilerParams(
            use_tc_tiling_on_sc=True, needs_layout_passes=False
        ),
    )(_tri_body)
    return fn(x, idx)


def kernel(input):
    idx = jnp.asarray(_IDX)
    return _tritovec(input, idx).reshape(_BATCH, _NNZ, 1)


# speedup vs baseline: 1.8793x; 1.8793x over previous
import functools

import jax
import jax.numpy as jnp
import numpy as np
from jax import lax
from jax.experimental import pallas as pl
from jax.experimental.pallas import tpu as pltpu
from jax.experimental.pallas import tpu_sc as plsc

_DIM = 256
_NNZ = _DIM * (_DIM + 1) // 2
_BATCH = 1024
_NTILES = 32
_PER_W = _BATCH // _NTILES
_HALF = _DIM // 2
_CNT_A = sum(_DIM - i for i in range(_HALF))
_CNT_B = _NNZ - _CNT_A


def _packed_triu_idx() -> np.ndarray:
    i, j = np.triu_indices(_DIM)
    a = i < _HALF
    idx_a = (i[a] << 8) | j[a]
    idx_b = ((i[~a] - _HALF) << 8) | (j[~a] - _HALF)
    return np.concatenate([idx_a, idx_b]).astype(np.int32)


_IDX = _packed_triu_idx()


def _tri_body(
    x_hbm, idx_hbm, out_hbm,
    idx_v, buf_a, buf_b, out_v,
    sem_a, sem_b, sem_out, sem_out2,
):
    nc = 2
    wid = lax.axis_index("s") * nc + lax.axis_index("c")
    b0 = wid * _PER_W

    def stage_a(b):
        return pltpu.make_async_copy(
            x_hbm.at[b, pl.ds(0, _HALF)], buf_a, sem_a
        )

    def stage_b(b):
        return pltpu.make_async_copy(
            x_hbm.at[b, pl.ds(_HALF, _HALF), pl.ds(_HALF, _HALF)],
            buf_b,
            sem_b,
        )

    def out_copy_a(b):
        return pltpu.make_async_copy(
            out_v.at[pl.ds(0, _CNT_A)],
            out_hbm.at[pl.ds(b * _NNZ, _CNT_A)],
            sem_out,
        )

    def out_copy_b(b):
        return pltpu.make_async_copy(
            out_v.at[pl.ds(_CNT_A, _CNT_B)],
            out_hbm.at[pl.ds(b * _NNZ + _CNT_A, _CNT_B)],
            sem_out2,
        )

    def gather_span(buf, chunk0, nchunks):
        group = 4

        @plsc.parallel_loop(0, nchunks // group, unroll=2)
        def _(cg):
            base = (chunk0 + cg * group) * 16
            ivs = [idx_v[pl.ds(base + 16 * u, 16)] for u in range(group)]
            rs = [lax.shift_right_logical(iv, 8) for iv in ivs]
            cs = [lax.bitwise_and(iv, 255) for iv in ivs]
            gs = [
                plsc.load_gather(buf, [rs[u], cs[u]]) for u in range(group)
            ]
            for u in range(group):
                out_v[pl.ds(base + 16 * u, 16)] = gs[u]

    stage_a(b0).start()
    stage_b(b0).start()
    pltpu.sync_copy(idx_hbm, idx_v)

    def batch_body(bl, carry):
        b = b0 + bl
        stage_a(b).wait()

        @pl.when(bl > 0)
        def _():
            out_copy_a(b - 1).wait()

        gather_span(buf_a, 0, _CNT_A // 16)

        @pl.when(bl < _PER_W - 1)
        def _():
            stage_a(b + 1).start()

        out_copy_a(b).start()
        stage_b(b).wait()

        @pl.when(bl > 0)
        def _():
            out_copy_b(b - 1).wait()

        gather_span(buf_b, _CNT_A // 16, _CNT_B // 16)

        @pl.when(bl < _PER_W - 1)
        def _():
            stage_b(b + 1).start()

        out_copy_b(b).start()
        return carry

    lax.fori_loop(0, _PER_W, batch_body, 0)
    out_copy_a(b0 + _PER_W - 1).wait()
    out_copy_b(b0 + _PER_W - 1).wait()


@jax.jit
def _tritovec(x, idx):
    mesh = plsc.VectorSubcoreMesh(core_axis_name="c", subcore_axis_name="s")
    fn = functools.partial(
        pl.kernel,
        mesh=mesh,
        out_type=jax.ShapeDtypeStruct((_BATCH * _NNZ,), jnp.float32),
        scratch_types=[
            pltpu.VMEM((_NNZ,), jnp.int32),
            pltpu.VMEM((_HALF, _DIM), jnp.float32),
            pltpu.VMEM((_HALF, _HALF), jnp.float32),
            pltpu.VMEM((_NNZ,), jnp.float32),
            pltpu.SemaphoreType.DMA,
            pltpu.SemaphoreType.DMA,
            pltpu.SemaphoreType.DMA,
            pltpu.SemaphoreType.DMA,
        ],
        compiler_params=pltpu.CompilerParams(
            use_tc_tiling_on_sc=True, needs_layout_passes=False
        ),
    )(_tri_body)
    return fn(x, idx)


def kernel(input):
    idx = jnp.asarray(_IDX)
    return _tritovec(input, idx).reshape(_BATCH, _NNZ, 1)


# speedup vs baseline: 1.9909x; 1.0594x over previous
import functools

import jax
import jax.numpy as jnp
import numpy as np
from jax import lax
from jax.experimental import pallas as pl
from jax.experimental.pallas import tpu as pltpu
from jax.experimental.pallas import tpu_sc as plsc

_DIM = 256
_NNZ = _DIM * (_DIM + 1) // 2
_BATCH = 1024
_NTILES = 32
_PER_W = _BATCH // _NTILES
_HALF = _DIM // 2
_CNT_A = sum(_DIM - i for i in range(_HALF))
_CNT_B = _NNZ - _CNT_A
_WORDS_A = _HALF * _DIM
_WORDS_B = _HALF * _HALF


def _flat_triu_idx() -> np.ndarray:
    i, j = np.triu_indices(_DIM)
    a = i < _HALF
    off_a = (
        ((i[a] >> 3) * 2 + (j[a] >> 7)) * 1024
        + (i[a] & 7) * 128
        + (j[a] & 127)
    )
    li, lj = i[~a] - _HALF, j[~a] - _HALF
    off_b = (li >> 3) * 1024 + (li & 7) * 128 + lj
    return np.concatenate([off_a, off_b]).astype(np.int32)


_IDX = _flat_triu_idx()


def _tri_body(
    x_hbm, idx_hbm, out_hbm,
    idx_v, buf_a, buf_b, out_v,
    sem_a, sem_b, sem_out, sem_out2,
):
    nc = 2
    wid = lax.axis_index("s") * nc + lax.axis_index("c")
    b0 = wid * _PER_W

    def stage_a(b):
        return pltpu.make_async_copy(
            x_hbm.at[b, pl.ds(0, _WORDS_A)], buf_a, sem_a
        )

    def stage_b_part(b, k):
        return pltpu.make_async_copy(
            x_hbm.at[b, pl.ds((16 + k) * 2048 + 1024, 1024)],
            buf_b.at[pl.ds(k * 1024, 1024)],
            sem_b,
        )

    def stage_b_start(b):
        for k in range(16):
            stage_b_part(b, k).start()

    def stage_b_wait(b):
        for k in range(16):
            stage_b_part(b, k).wait()

    def out_copy_a(b):
        return pltpu.make_async_copy(
            out_v.at[pl.ds(0, _CNT_A)],
            out_hbm.at[pl.ds(b * _NNZ, _CNT_A)],
            sem_out,
        )

    def out_copy_b(b):
        return pltpu.make_async_copy(
            out_v.at[pl.ds(_CNT_A, _CNT_B)],
            out_hbm.at[pl.ds(b * _NNZ + _CNT_A, _CNT_B)],
            sem_out2,
        )

    def gather_span(buf, chunk0, nchunks):
        group = 4

        @plsc.parallel_loop(0, nchunks // group, unroll=4)
        def _(cg):
            base = (chunk0 + cg * group) * 16
            ivs = [idx_v[pl.ds(base + 16 * u, 16)] for u in range(group)]
            gs = [plsc.load_gather(buf, [iv]) for iv in ivs]
            for u in range(group):
                out_v[pl.ds(base + 16 * u, 16)] = gs[u]

    stage_a(b0).start()
    stage_b_start(b0)
    pltpu.sync_copy(idx_hbm, idx_v)

    def batch_body(bl, carry):
        b = b0 + bl
        stage_a(b).wait()

        @pl.when(bl > 0)
        def _():
            out_copy_a(b - 1).wait()

        gather_span(buf_a, 0, _CNT_A // 16)

        @pl.when(bl < _PER_W - 1)
        def _():
            stage_a(b + 1).start()

        out_copy_a(b).start()
        stage_b_wait(b)

        @pl.when(bl > 0)
        def _():
            out_copy_b(b - 1).wait()

        gather_span(buf_b, _CNT_A // 16, _CNT_B // 16)

        @pl.when(bl < _PER_W - 1)
        def _():
            stage_b_start(b + 1)

        out_copy_b(b).start()
        return carry

    lax.fori_loop(0, _PER_W, batch_body, 0)
    out_copy_a(b0 + _PER_W - 1).wait()
    out_copy_b(b0 + _PER_W - 1).wait()


@jax.jit
def _tritovec(x, idx):
    mesh = plsc.VectorSubcoreMesh(core_axis_name="c", subcore_axis_name="s")
    fn = functools.partial(
        pl.kernel,
        mesh=mesh,
        out_type=jax.ShapeDtypeStruct((_BATCH * _NNZ,), jnp.float32),
        scratch_types=[
            pltpu.VMEM((_NNZ,), jnp.int32),
            pltpu.VMEM((_WORDS_A,), jnp.float32),
            pltpu.VMEM((_WORDS_B,), jnp.float32),
            pltpu.VMEM((_NNZ,), jnp.float32),
            pltpu.SemaphoreType.DMA,
            pltpu.SemaphoreType.DMA,
            pltpu.SemaphoreType.DMA,
            pltpu.SemaphoreType.DMA,
        ],
        compiler_params=pltpu.CompilerParams(
            use_tc_tiling_on_sc=False, needs_layout_passes=False
        ),
    )(_tri_body)
    return fn(x, idx)


def kernel(input):
    idx = jnp.asarray(_IDX)
    x5 = input.reshape(_BATCH, 32, 8, 2, 128).transpose(0, 1, 3, 2, 4)
    x2 = x5.reshape(_BATCH, 65536)
    return _tritovec(x2, idx).reshape(_BATCH, _NNZ, 1)


# speedup vs baseline: 2.1437x; 1.0768x over previous
import functools

import jax
import jax.numpy as jnp
import numpy as np
from jax import lax
from jax.experimental import pallas as pl
from jax.experimental.pallas import tpu as pltpu
from jax.experimental.pallas import tpu_sc as plsc

_DIM = 256
_NNZ = _DIM * (_DIM + 1) // 2
_BATCH = 1024
_NTILES = 32
_PER_W = _BATCH // _NTILES
_HALF = _DIM // 2
_CNT_A = sum(_DIM - i for i in range(_HALF))
_CNT_B = _NNZ - _CNT_A
_WORDS_A = _HALF * _DIM
_WORDS_B = _HALF * _HALF


def _flat_triu_idx() -> np.ndarray:
    i, j = np.triu_indices(_DIM)
    a = i < _HALF
    off_a = (
        ((i[a] >> 3) * 2 + (j[a] >> 7)) * 1024
        + (i[a] & 7) * 128
        + (j[a] & 127)
    )
    li, lj = i[~a] - _HALF, j[~a] - _HALF
    off_b = (li >> 3) * 1024 + (li & 7) * 128 + lj
    idx = np.concatenate([off_a, off_b]).astype(np.int32)
    pairs = idx.reshape(-1, 2, 16)
    return (pairs[:, 0, :] | (pairs[:, 1, :] << 16)).reshape(-1)


_IDX = _flat_triu_idx()


def _tri_body(
    x_hbm, idx_hbm, out_hbm,
    idx_v, buf_a, buf_b, out_v,
    sem_a, sem_b, sem_out, sem_out2,
):
    nc = 2
    wid = lax.axis_index("s") * nc + lax.axis_index("c")
    b0 = wid * _PER_W

    def stage_a(b):
        return pltpu.make_async_copy(
            x_hbm.at[b, pl.ds(0, _WORDS_A)], buf_a, sem_a
        )

    def stage_b_part(b, k):
        return pltpu.make_async_copy(
            x_hbm.at[b, pl.ds((16 + k) * 2048 + 1024, 1024)],
            buf_b.at[pl.ds(k * 1024, 1024)],
            sem_b,
        )

    def stage_b_start(b):
        for k in range(16):
            stage_b_part(b, k).start()

    def stage_b_wait(b):
        for k in range(16):
            stage_b_part(b, k).wait()

    def out_copy_a(b):
        return pltpu.make_async_copy(
            out_v.at[pl.ds(0, _CNT_A)],
            out_hbm.at[pl.ds(b * _NNZ, _CNT_A)],
            sem_out,
        )

    def out_copy_b(b):
        return pltpu.make_async_copy(
            out_v.at[pl.ds(_CNT_A, _CNT_B)],
            out_hbm.at[pl.ds(b * _NNZ + _CNT_A, _CNT_B)],
            sem_out2,
        )

    def gather_span(buf, pair0, npairs):
        group = 2

        @plsc.parallel_loop(0, npairs // group, unroll=4)
        def _(cg):
            pbase = pair0 + cg * group
            ivs = [idx_v[pl.ds((pbase + u) * 16, 16)] for u in range(group)]
            los = [lax.bitwise_and(iv, 0xFFFF) for iv in ivs]
            his = [lax.shift_right_logical(iv, 16) for iv in ivs]
            gs = []
            for u in range(group):
                gs.append(plsc.load_gather(buf, [los[u]]))
                gs.append(plsc.load_gather(buf, [his[u]]))
            for u in range(group):
                obase = (pbase + u) * 32
                out_v[pl.ds(obase, 16)] = gs[2 * u]
                out_v[pl.ds(obase + 16, 16)] = gs[2 * u + 1]

    stage_a(b0).start()
    stage_b_start(b0)
    pltpu.sync_copy(idx_hbm, idx_v)

    def batch_body(bl, carry):
        b = b0 + bl
        stage_a(b).wait()

        @pl.when(bl > 0)
        def _():
            out_copy_a(b - 1).wait()

        gather_span(buf_a, 0, _CNT_A // 32)

        @pl.when(bl < _PER_W - 1)
        def _():
            stage_a(b + 1).start()

        out_copy_a(b).start()
        stage_b_wait(b)

        @pl.when(bl > 0)
        def _():
            out_copy_b(b - 1).wait()

        gather_span(buf_b, _CNT_A // 32, _CNT_B // 32)

        @pl.when(bl < _PER_W - 1)
        def _():
            stage_b_start(b + 1)

        out_copy_b(b).start()
        return carry

    lax.fori_loop(0, _PER_W, batch_body, 0)
    out_copy_a(b0 + _PER_W - 1).wait()
    out_copy_b(b0 + _PER_W - 1).wait()


@jax.jit
def _tritovec(x, idx):
    mesh = plsc.VectorSubcoreMesh(core_axis_name="c", subcore_axis_name="s")
    fn = functools.partial(
        pl.kernel,
        mesh=mesh,
        out_type=jax.ShapeDtypeStruct((_BATCH * _NNZ,), jnp.float32),
        scratch_types=[
            pltpu.VMEM((_NNZ // 2,), jnp.int32),
            pltpu.VMEM((_WORDS_A,), jnp.float32),
            pltpu.VMEM((_WORDS_B,), jnp.float32),
            pltpu.VMEM((_NNZ,), jnp.float32),
            pltpu.SemaphoreType.DMA,
            pltpu.SemaphoreType.DMA,
            pltpu.SemaphoreType.DMA,
            pltpu.SemaphoreType.DMA,
        ],
        compiler_params=pltpu.CompilerParams(
            use_tc_tiling_on_sc=False, needs_layout_passes=False
        ),
    )(_tri_body)
    return fn(x, idx)


def kernel(input):
    idx = jnp.asarray(_IDX)
    x5 = input.reshape(_BATCH, 32, 8, 2, 128).transpose(0, 1, 3, 2, 4)
    x2 = x5.reshape(_BATCH, 65536)
    return _tritovec(x2, idx).reshape(_BATCH, _NNZ, 1)
